# Initial kernel scaffold; baseline (speedup 1.0000x reference)
import functools

import jax
import jax.numpy as jnp
from jax import lax
from jax.experimental import pallas as pl
from jax.experimental.pallas import tpu as pltpu
from jax.experimental.pallas import tpu_sc as plsc

NC = 2
NS = 16
NW = NC * NS

CHUNK = 128


def _body(n_chunks, src_r, f0i_r, f1i_r, f2i_r, char_r, t0_r, t1_r, t2_r,
          out_r, idx_v, fidx_v, char_v, f_v, sem):
    wid = lax.axis_index("s") * NC + lax.axis_index("c")

    pltpu.sync_copy(src_r.at[wid], idx_v)
    pltpu.sync_copy(f0i_r.at[wid], fidx_v.at[0])
    pltpu.sync_copy(f1i_r.at[wid], fidx_v.at[1])
    pltpu.sync_copy(f2i_r.at[wid], fidx_v.at[2])

    def chunk_body(j, carry):
        c0 = pltpu.async_copy(char_r.at[idx_v.at[j]], char_v, sem)
        c1 = pltpu.async_copy(t0_r.at[fidx_v.at[0, j]], f_v.at[0], sem)
        c2 = pltpu.async_copy(t1_r.at[fidx_v.at[1, j]], f_v.at[1], sem)
        c3 = pltpu.async_copy(t2_r.at[fidx_v.at[2, j]], f_v.at[2], sem)
        c0.wait()
        c1.wait()
        c2.wait()
        c3.wait()
        base = (wid * n_chunks + j) * CHUNK
        rows = pl.ds(base, CHUNK)
        pltpu.sync_copy(char_v, out_r.at[rows, pl.ds(0, 64)])
        pltpu.sync_copy(f_v.at[0], out_r.at[rows, pl.ds(64, 16)])
        pltpu.sync_copy(f_v.at[1], out_r.at[rows, pl.ds(80, 16)])
        pltpu.sync_copy(f_v.at[2], out_r.at[rows, pl.ds(96, 16)])
        return carry

    lax.fori_loop(0, n_chunks, chunk_body, 0)


def kernel(src, feats, char_table, feat_tables):
    B, L = src.shape
    N = B * L
    assert N % (NW * CHUNK) == 0
    n_chunks = N // (NW * CHUNK)
    D_char = char_table.shape[1]
    D_feat = feat_tables.shape[2]
    D = D_char + 3 * D_feat

    src_w = src.reshape(NW, n_chunks, CHUNK).astype(jnp.int32)
    feats_w = feats.reshape(3, NW, n_chunks, CHUNK).astype(jnp.int32)

    mesh = plsc.VectorSubcoreMesh(
        core_axis_name="c", subcore_axis_name="s",
        num_cores=NC, num_subcores=NS)

    run = pl.kernel(
        functools.partial(_body, n_chunks),
        out_type=jax.ShapeDtypeStruct((N, D), jnp.float32),
        mesh=mesh,
        scratch_types=[
            pltpu.VMEM((n_chunks, CHUNK), jnp.int32),
            pltpu.VMEM((3, n_chunks, CHUNK), jnp.int32),
            pltpu.VMEM((CHUNK, D_char), jnp.float32),
            pltpu.VMEM((3, CHUNK, D_feat), jnp.float32),
            pltpu.SemaphoreType.DMA,
        ],
    )
    out = run(src_w, feats_w[0], feats_w[1], feats_w[2], char_table,
              feat_tables[0], feat_tables[1], feat_tables[2])
    return out.reshape(B, L, D)


# speedup vs baseline: 8.3929x; 8.3929x over previous
import functools

import jax
import jax.numpy as jnp
from jax import lax
from jax.experimental import pallas as pl
from jax.experimental.pallas import tpu as pltpu
from jax.experimental.pallas import tpu_sc as plsc

NC = 2
NS = 16
NW = NC * NS

CHUNK = 128


def _body(n_chunks, src_r, f0i_r, f1i_r, f2i_r, char_r, t0_r, t1_r, t2_r,
          out_r, idx_v, fidx_v, char_v, f_v, sem):
    wid = lax.axis_index("s") * NC + lax.axis_index("c")

    pltpu.sync_copy(src_r.at[wid], idx_v)
    pltpu.sync_copy(f0i_r.at[wid], fidx_v.at[0])
    pltpu.sync_copy(f1i_r.at[wid], fidx_v.at[1])
    pltpu.sync_copy(f2i_r.at[wid], fidx_v.at[2])

    def chunk_body(j, carry):
        c0 = pltpu.async_copy(char_r.at[idx_v.at[j]], char_v, sem)
        c1 = pltpu.async_copy(t0_r.at[fidx_v.at[0, j]], f_v.at[0], sem)
        c2 = pltpu.async_copy(t1_r.at[fidx_v.at[1, j]], f_v.at[1], sem)
        c3 = pltpu.async_copy(t2_r.at[fidx_v.at[2, j]], f_v.at[2], sem)
        c0.wait()
        c1.wait()
        c2.wait()
        c3.wait()
        base = (wid * n_chunks + j) * CHUNK
        rows = pl.ds(base, CHUNK)
        pltpu.sync_copy(char_v, out_r.at[rows, pl.ds(0, 64)])
        pltpu.sync_copy(f_v.at[0], out_r.at[rows, pl.ds(64, 16)])
        pltpu.sync_copy(f_v.at[1], out_r.at[rows, pl.ds(80, 16)])
        pltpu.sync_copy(f_v.at[2], out_r.at[rows, pl.ds(96, 16)])
        return carry

    lax.fori_loop(0, n_chunks, chunk_body, 0)


def kernel(src, feats, char_table, feat_tables):
    B, L = src.shape
    N = B * L
    assert N % (NW * CHUNK) == 0
    n_chunks = N // (NW * CHUNK)
    D_char = char_table.shape[1]
    D_feat = feat_tables.shape[2]
    D = D_char + 3 * D_feat

    src_w = src.reshape(NW, n_chunks, CHUNK).astype(jnp.int32)
    feats_w = feats.reshape(3, NW, n_chunks, CHUNK).astype(jnp.int32)

    mesh = plsc.VectorSubcoreMesh(
        core_axis_name="c", subcore_axis_name="s",
        num_cores=NC, num_subcores=NS)

    run = pl.kernel(
        functools.partial(_body, n_chunks),
        out_type=jax.ShapeDtypeStruct((N, D), jnp.float32),
        mesh=mesh,
        scratch_types=[
            pltpu.VMEM((n_chunks, CHUNK), jnp.int32),
            pltpu.VMEM((3, n_chunks, CHUNK), jnp.int32),
            pltpu.VMEM((CHUNK, D_char), jnp.float32),
            pltpu.VMEM((3, CHUNK, D_feat), jnp.float32),
            pltpu.SemaphoreType.DMA,
        ],
        compiler_params=pltpu.CompilerParams(use_tc_tiling_on_sc=False),
    )
    out = run(src_w, feats_w[0], feats_w[1], feats_w[2], char_table,
              feat_tables[0], feat_tables[1], feat_tables[2])
    return out.reshape(B, L, D)


# speedup vs baseline: 9.2697x; 1.1045x over previous
import functools

import jax
import jax.numpy as jnp
from jax import lax
from jax.experimental import pallas as pl
from jax.experimental.pallas import tpu as pltpu
from jax.experimental.pallas import tpu_sc as plsc

NC = 2
NS = 16
NW = NC * NS

CHUNK = 128
NBUF = 5
DEPTH = 3


def _body(n_chunks, src_r, f0i_r, f1i_r, f2i_r, char_r, t0_r, t1_r, t2_r,
          out_r, idx_v, fidx_v, char_v, f_v, *sems):
    gsem = sems[:NBUF]
    wsem = sems[NBUF:]
    wid = lax.axis_index("s") * NC + lax.axis_index("c")

    pltpu.sync_copy(src_r.at[wid], idx_v)
    pltpu.sync_copy(f0i_r.at[wid], fidx_v.at[0])
    pltpu.sync_copy(f1i_r.at[wid], fidx_v.at[1])
    pltpu.sync_copy(f2i_r.at[wid], fidx_v.at[2])

    fidx = (fidx_v.at[0], fidx_v.at[1], fidx_v.at[2])
    tabs = (t0_r, t1_r, t2_r)

    def out_slices(q):
        rows = pl.ds(q * CHUNK, CHUNK)
        return (out_r.at[rows, pl.ds(0, 64)],
                out_r.at[rows, pl.ds(64, 16)],
                out_r.at[rows, pl.ds(80, 16)],
                out_r.at[rows, pl.ds(96, 16)])

    def bufs(b):
        return (char_v.at[b], f_v.at[b, 0], f_v.at[b, 1], f_v.at[b, 2])

    def fire_gathers(q, b):
        pltpu.async_copy(char_r.at[idx_v.at[q]], char_v.at[b], gsem[b])
        for i in range(3):
            pltpu.async_copy(tabs[i].at[fidx[i].at[q]], f_v.at[b, i], gsem[b])

    def wait_gathers(b):
        dsts = bufs(b)
        for i, src in enumerate(out_slices(0)):
            pltpu.make_async_copy(src, dsts[i], gsem[b]).wait()

    def fire_writes(q, b):
        srcs = bufs(b)
        for i, dst in enumerate(out_slices(wid * n_chunks + q)):
            pltpu.async_copy(srcs[i], dst, wsem[b])

    def drain_writes(b):
        srcs = bufs(b)
        for i, dst in enumerate(out_slices(0)):
            pltpu.make_async_copy(srcs[i], dst, wsem[b]).wait()

    for q in range(DEPTH):
        fire_gathers(q, q)

    def outer_body(g, carry):
        for b in range(NBUF):
            j = g * NBUF + b
            bd = (b + DEPTH) % NBUF

            @pl.when(j + DEPTH < n_chunks)
            def _():
                @pl.when(j + DEPTH >= NBUF)
                def _():
                    drain_writes(bd)
                fire_gathers(j + DEPTH, bd)

            wait_gathers(b)
            fire_writes(j, b)
        return carry

    lax.fori_loop(0, n_chunks // NBUF, outer_body, 0)
    for b in range(NBUF):
        drain_writes(b)


def kernel(src, feats, char_table, feat_tables):
    B, L = src.shape
    N = B * L
    assert N % (NW * CHUNK) == 0
    n_chunks = N // (NW * CHUNK)
    assert n_chunks % NBUF == 0 and n_chunks >= NBUF + DEPTH
    D_char = char_table.shape[1]
    D_feat = feat_tables.shape[2]
    D = D_char + 3 * D_feat

    src_w = src.reshape(NW, n_chunks, CHUNK).astype(jnp.int32)
    feats_w = feats.reshape(3, NW, n_chunks, CHUNK).astype(jnp.int32)

    mesh = plsc.VectorSubcoreMesh(
        core_axis_name="c", subcore_axis_name="s",
        num_cores=NC, num_subcores=NS)

    run = pl.kernel(
        functools.partial(_body, n_chunks),
        out_type=jax.ShapeDtypeStruct((N, D), jnp.float32),
        mesh=mesh,
        scratch_types=[
            pltpu.VMEM((n_chunks, CHUNK), jnp.int32),
            pltpu.VMEM((3, n_chunks, CHUNK), jnp.int32),
            pltpu.VMEM((NBUF, CHUNK, D_char), jnp.float32),
            pltpu.VMEM((NBUF, 3, CHUNK, D_feat), jnp.float32),
        ] + [pltpu.SemaphoreType.DMA] * (2 * NBUF),
        compiler_params=pltpu.CompilerParams(use_tc_tiling_on_sc=False),
    )
    out = run(src_w, feats_w[0], feats_w[1], feats_w[2], char_table,
              feat_tables[0], feat_tables[1], feat_tables[2])
    return out.reshape(B, L, D)


# speedup vs baseline: 9.4941x; 1.0242x over previous
import functools

import jax
import jax.numpy as jnp
from jax import lax
from jax.experimental import pallas as pl
from jax.experimental.pallas import tpu as pltpu
from jax.experimental.pallas import tpu_sc as plsc

NC = 2
NS = 16
NW = NC * NS

NBUF = 8
DEPTH = 5


def _body(b_per_w, L, src_r, fci_r, char_r, tabc_r,
          out_r, idx_v, fidx_v, char_v, fc_v, *sems):
    gsem = sems[:NBUF]
    wsem = sems[NBUF:]
    wid = lax.axis_index("s") * NC + lax.axis_index("c")
    wb = wid * b_per_w

    pltpu.sync_copy(src_r.at[pl.ds(wb, b_per_w), :], idx_v)
    pltpu.sync_copy(fci_r.at[pl.ds(wb, b_per_w), :], fidx_v)

    def out_slices(b0):
        return (out_r.at[b0, :, pl.ds(0, 64)],
                out_r.at[b0, :, pl.ds(64, 16)],
                out_r.at[b0, :, pl.ds(80, 16)],
                out_r.at[b0, :, pl.ds(96, 16)])

    def bufs(b):
        return (char_v.at[b],
                fc_v.at[b, pl.ds(0, L)],
                fc_v.at[b, pl.ds(L, L)],
                fc_v.at[b, pl.ds(2 * L, L)])

    def fire_gathers(q, b):
        pltpu.async_copy(char_r.at[idx_v.at[q, :]], char_v.at[b], gsem[b])
        pltpu.async_copy(tabc_r.at[fidx_v.at[q, :]], fc_v.at[b], gsem[b])

    def wait_gathers(b):
        pltpu.make_async_copy(out_r.at[0, :, pl.ds(0, 64)],
                              char_v.at[b], gsem[b]).wait()
        pltpu.make_async_copy(tabc_r.at[pl.ds(0, 3 * L), :],
                              fc_v.at[b], gsem[b]).wait()

    def fire_writes(q, b):
        srcs = bufs(b)
        for i, dst in enumerate(out_slices(wb + q)):
            pltpu.async_copy(srcs[i], dst, wsem[b])

    def drain_writes(b):
        srcs = bufs(b)
        for i, dst in enumerate(out_slices(0)):
            pltpu.make_async_copy(srcs[i], dst, wsem[b]).wait()

    for q in range(DEPTH):
        fire_gathers(q, q)

    def outer_body(g, carry):
        for b in range(NBUF):
            j = g * NBUF + b
            bd = (b + DEPTH) % NBUF

            @pl.when(j + DEPTH < b_per_w)
            def _():
                @pl.when(j + DEPTH >= NBUF)
                def _():
                    drain_writes(bd)
                fire_gathers(j + DEPTH, bd)

            wait_gathers(b)
            fire_writes(j, b)
        return carry

    lax.fori_loop(0, b_per_w // NBUF, outer_body, 0)
    for b in range(NBUF):
        drain_writes(b)


def kernel(src, feats, char_table, feat_tables):
    B, L = src.shape
    assert B % NW == 0
    b_per_w = B // NW
    assert b_per_w % NBUF == 0 and b_per_w >= NBUF + DEPTH
    F, V_f, D_feat = feat_tables.shape
    D_char = char_table.shape[1]
    D = D_char + F * D_feat

    src_i = src.astype(jnp.int32)
    feats_i = feats.astype(jnp.int32) + (jnp.arange(F, dtype=jnp.int32)
                                         * V_f)[:, None, None]
    fci = jnp.concatenate([feats_i[i] for i in range(F)], axis=1)
    tab_c = feat_tables.reshape(F * V_f, D_feat)

    mesh = plsc.VectorSubcoreMesh(
        core_axis_name="c", subcore_axis_name="s",
        num_cores=NC, num_subcores=NS)

    run = pl.kernel(
        functools.partial(_body, b_per_w, L),
        out_type=jax.ShapeDtypeStruct((B, L, D), jnp.float32),
        mesh=mesh,
        scratch_types=[
            pltpu.VMEM((b_per_w, L), jnp.int32),
            pltpu.VMEM((b_per_w, F * L), jnp.int32),
            pltpu.VMEM((NBUF, L, D_char), jnp.float32),
            pltpu.VMEM((NBUF, F * L, D_feat), jnp.float32),
        ] + [pltpu.SemaphoreType.DMA] * (2 * NBUF),
        compiler_params=pltpu.CompilerParams(use_tc_tiling_on_sc=False),
    )
    return run(src_i, fci, char_table, tab_c)
